# baseline (speedup 1.0000x reference)
import jax
import jax.numpy as jnp
from jax.experimental import pallas as pl


_BLK = 256


def _body(w_ref, o_ref):
    o_ref[...] = jnp.broadcast_to(w_ref[...][None], o_ref.shape)


def kernel(x, pe_weight):
    batch = x.shape[0]
    max_len, d_model = pe_weight.shape
    return pl.pallas_call(
        _body,
        grid=(max_len // _BLK,),
        in_specs=[pl.BlockSpec((_BLK, d_model), lambda i: (i, 0))],
        out_specs=pl.BlockSpec((batch, _BLK, d_model), lambda i: (0, i, 0)),
        out_shape=jax.ShapeDtypeStruct((batch, max_len, d_model), pe_weight.dtype),
    )(pe_weight)
